# baseline (speedup 1.0000x reference)
import jax
import jax.numpy as jnp
from jax import lax
from jax.experimental import pallas as pl
from jax.experimental.pallas import tpu as pltpu
from jax.experimental.pallas import tpu_sc as plsc

NA = 6_400_000
NM = 100_000
NMP = 100_352
NC = 2
NS = 16
APW = NA // (NC * NS)
PIECE = 4_000
NPIECE = APW // PIECE
NSTR = 28
NTEC = NPIECE - NSTR
NROUND = max(NSTR, NTEC)
ZCH = NMP // NS
CCH = NMP // 32
WCH = 4
NV = PIECE // 16


def _partials_kernel(idx_hbm, val_hbm, part_hbm,
                     tidx0, tval0, tidx1, tval1,
                     sidx0, sval0, sidx1, sval1, sidx2, sval2, sidx3, sval3,
                     iotabuf, t16, acc_v, acc,
                     tl0, tl1, sl0, sl1, sl2, sl3, sc0, sc1, sc2, sc3):
    c = lax.axis_index("c")
    s = lax.axis_index("s")
    wid = c * NS + s
    tbufs = ((tidx0, tval0, tl0), (tidx1, tval1, tl1))
    sbufs = ((sidx0, sval0, sl0, sc0), (sidx1, sval1, sl1, sc1),
             (sidx2, sval2, sl2, sc2), (sidx3, sval3, sl3, sc3))

    def start_load(i, idxb, valb, sem):
        base = wid * APW + i * PIECE
        pltpu.async_copy(idx_hbm.at[pl.ds(base, PIECE)], idxb, sem)
        pltpu.async_copy(val_hbm.at[pl.ds(base, PIECE)], valb, sem)

    def wait_load(idxb, valb, sem):
        pltpu.make_async_copy(idx_hbm.at[pl.ds(0, PIECE)], idxb, sem).wait()
        pltpu.make_async_copy(val_hbm.at[pl.ds(0, PIECE)], valb, sem).wait()

    def zero_body(j, _):
        acc_v[pl.ds(16 * j, 16)] = jnp.zeros((16,), jnp.float32)
        return _

    lax.fori_loop(0, WCH * CCH // 16, zero_body, None)
    pltpu.sync_copy(acc_v.at[pl.ds(0, ZCH)], acc.at[pl.ds(s * ZCH, ZCH)])

    start_load(0, sidx0, sval0, sl0)
    start_load(1, sidx1, sval1, sl1)
    start_load(NSTR + 0, tidx0, tval0, tl0)
    start_load(NSTR + 1, tidx1, tval1, tl1)

    pltpu.sync_copy(idx_hbm.at[pl.ds(wid * APW, 16)], t16)
    m_first = t16[...][0]
    pltpu.sync_copy(idx_hbm.at[pl.ds(wid * APW + APW - 16, 16)], t16)
    m_last = t16[...][15]
    k_lo = m_first // CCH
    k_hi = m_last // CCH
    fits = (k_hi - k_lo) < WCH
    win_base = k_lo * CCH

    iota16 = lax.iota(jnp.int32, 16)
    prev_sel = jnp.maximum(iota16 - 1, 0)
    next_clamp = jnp.minimum(iota16 + (PIECE - 16) + 1, PIECE - 1)
    lane0 = iota16 == 0
    lane15 = iota16 == 15

    def vreg_step(iv, vv, iv_prev, iv_next):
        c0 = plsc.cumsum(vv)
        cx = c0 - vv
        last = (iv != iv_next) | lane15
        restart = (iv != iv_prev) | lane0
        ivl = iv - win_base
        plsc.addupdate_scatter(acc_v, [ivl], c0, mask=last)
        plsc.addupdate_scatter(acc_v, [ivl], -cx, mask=restart)

    def compute_piece(idxb, valb):
        iv = idxb[pl.ds(0, 16)]
        vreg_step(iv, valb[pl.ds(0, 16)],
                  plsc.load_gather(idxb, [prev_sel]), idxb[pl.ds(1, 16)])

        def vb(j, _):
            for u in range(4):
                b = 64 * j + 16 * u + 16
                vreg_step(idxb[pl.ds(b, 16)], valb[pl.ds(b, 16)],
                          idxb[pl.ds(b - 1, 16)], idxb[pl.ds(b + 1, 16)])
            return _

        lax.fori_loop(0, (NV - 2) // 4, vb, None)

        b = PIECE - 16
        vreg_step(idxb[pl.ds(b, 16)], valb[pl.ds(b, 16)],
                  idxb[pl.ds(b - 1, 16)],
                  plsc.load_gather(idxb, [next_clamp]))

    def round_step(k, _):
        for t in range(2):
            idxb, valb, sem = tbufs[t]

            @pl.when((k < NTEC) & (k % 2 == t))
            def _():
                wait_load(idxb, valb, sem)

                @pl.when(fits)
                def _():
                    compute_piece(idxb, valb)

                @pl.when(jnp.logical_not(fits))
                def _():
                    pltpu.sync_copy(valb, acc.at[idxb], add=True)

                @pl.when(k + 2 < NTEC)
                def _():
                    start_load(NSTR + k + 2, idxb, valb, sem)

        for t in range(4):
            idxb, valb, lsem, csem = sbufs[t]

            @pl.when((k < NSTR) & (k % 4 == t))
            def _():
                wait_load(idxb, valb, lsem)
                pltpu.async_copy(valb, acc.at[idxb], csem, add=True)

            @pl.when((k + 2 < NSTR) & ((k + 2) % 4 == t))
            def _():
                @pl.when(k >= 2)
                def _():
                    pltpu.make_async_copy(valb, acc.at[idxb], csem).wait()

                start_load(k + 2, idxb, valb, lsem)

        return _

    lax.fori_loop(0, NROUND, round_step, None)

    for j in range(NSTR - 4, NSTR):
        idxb, valb, _, csem = sbufs[j % 4]
        pltpu.make_async_copy(valb, acc.at[idxb], csem).wait()

    plsc.subcore_barrier()

    for r in range(WCH):
        @pl.when(fits & (k_lo + r <= k_hi))
        def _():
            def ib(i, _):
                iotabuf[pl.ds(16 * i, 16)] = (iota16 + win_base
                                              + (r * CCH + 16 * i))
                return _

            lax.fori_loop(0, CCH // 16, ib, None)
            pltpu.sync_copy(acc_v.at[pl.ds(r * CCH, CCH)], acc.at[iotabuf],
                            add=True)

    plsc.subcore_barrier()

    pltpu.sync_copy(acc.at[pl.ds(s * ZCH, ZCH)],
                    part_hbm.at[pl.ds(c * NMP + s * ZCH, ZCH)])


def _combine_kernel(part_hbm, out_hbm, bufa, bufb):
    c = lax.axis_index("c")
    s = lax.axis_index("s")
    w = c * NS + s
    ch = NMP // (NC * NS)
    base = w * ch
    pltpu.sync_copy(part_hbm.at[pl.ds(base, ch)], bufa)
    pltpu.sync_copy(part_hbm.at[pl.ds(NMP + base, ch)], bufb)

    def add_body(j, _):
        sl = pl.ds(16 * j, 16)
        bufa[sl] = bufa[sl] + bufb[sl]
        return _

    lax.fori_loop(0, ch // 16, add_body, None)
    pltpu.sync_copy(bufa, out_hbm.at[pl.ds(base, ch)])


def kernel(indices, per_atom_property):
    mesh = plsc.VectorSubcoreMesh(core_axis_name="c", subcore_axis_name="s")

    partials = pl.kernel(
        _partials_kernel,
        out_type=jax.ShapeDtypeStruct((NC * NMP,), jnp.float32),
        mesh=mesh,
        compiler_params=pltpu.CompilerParams(needs_layout_passes=False),
        scratch_types=(
            [pltpu.VMEM((PIECE,), jnp.int32), pltpu.VMEM((PIECE,), jnp.float32)] * 6
            + [
                pltpu.VMEM((CCH,), jnp.int32),
                pltpu.VMEM((16,), jnp.int32),
                pltpu.VMEM((WCH * CCH,), jnp.float32),
                pltpu.VMEM_SHARED((NMP,), jnp.float32),
            ]
            + [pltpu.SemaphoreType.DMA] * 10
        ),
    )(indices, per_atom_property)

    out = pl.kernel(
        _combine_kernel,
        out_type=jax.ShapeDtypeStruct((NMP,), jnp.float32),
        mesh=mesh,
        scratch_types=[
            pltpu.VMEM((NMP // (NC * NS),), jnp.float32),
            pltpu.VMEM((NMP // (NC * NS),), jnp.float32),
        ],
    )(partials)

    return out[:NM]


# speedup vs baseline: 1.0494x; 1.0494x over previous
import jax
import jax.numpy as jnp
from jax import lax
from jax.experimental import pallas as pl
from jax.experimental.pallas import tpu as pltpu
from jax.experimental.pallas import tpu_sc as plsc

NA = 6_400_000
NM = 100_000
NMP = 100_352
NC = 2
NS = 16
APW = NA // (NC * NS)
PIECE = 4_000
NPIECE = APW // PIECE
NSTR = 25
NTEC = NPIECE - NSTR
NROUND = max(NSTR, NTEC)
ZCH = NMP // NS
CCH = NMP // 32
WCH = 4
NV = PIECE // 16


def _partials_kernel(idx_hbm, val_hbm, part_hbm,
                     tidx0, tval0, tidx1, tval1,
                     sidx0, sval0, sidx1, sval1, sidx2, sval2, sidx3, sval3,
                     iotabuf, t16, acc_v, acc,
                     tl0, tl1, sl0, sl1, sl2, sl3, sc0, sc1, sc2, sc3):
    c = lax.axis_index("c")
    s = lax.axis_index("s")
    wid = c * NS + s
    tbufs = ((tidx0, tval0, tl0), (tidx1, tval1, tl1))
    sbufs = ((sidx0, sval0, sl0, sc0), (sidx1, sval1, sl1, sc1),
             (sidx2, sval2, sl2, sc2), (sidx3, sval3, sl3, sc3))

    def start_load(i, idxb, valb, sem):
        base = wid * APW + i * PIECE
        pltpu.async_copy(idx_hbm.at[pl.ds(base, PIECE)], idxb, sem)
        pltpu.async_copy(val_hbm.at[pl.ds(base, PIECE)], valb, sem)

    def wait_load(idxb, valb, sem):
        pltpu.make_async_copy(idx_hbm.at[pl.ds(0, PIECE)], idxb, sem).wait()
        pltpu.make_async_copy(val_hbm.at[pl.ds(0, PIECE)], valb, sem).wait()

    def zero_body(j, _):
        acc_v[pl.ds(16 * j, 16)] = jnp.zeros((16,), jnp.float32)
        return _

    lax.fori_loop(0, WCH * CCH // 16, zero_body, None)
    pltpu.sync_copy(acc_v.at[pl.ds(0, ZCH)], acc.at[pl.ds(s * ZCH, ZCH)])

    start_load(0, sidx0, sval0, sl0)
    start_load(1, sidx1, sval1, sl1)
    start_load(NSTR + 0, tidx0, tval0, tl0)
    start_load(NSTR + 1, tidx1, tval1, tl1)

    pltpu.sync_copy(idx_hbm.at[pl.ds(wid * APW, 16)], t16)
    m_first = t16[...][0]
    pltpu.sync_copy(idx_hbm.at[pl.ds(wid * APW + APW - 16, 16)], t16)
    m_last = t16[...][15]
    k_lo = m_first // CCH
    k_hi = m_last // CCH
    fits = (k_hi - k_lo) < WCH
    win_base = k_lo * CCH

    iota16 = lax.iota(jnp.int32, 16)
    prev_sel = jnp.maximum(iota16 - 1, 0)
    next_clamp = jnp.minimum(iota16 + (PIECE - 16) + 1, PIECE - 1)
    lane0 = iota16 == 0
    lane15 = iota16 == 15

    def vreg_step(iv, vv, iv_prev, iv_next):
        c0 = plsc.cumsum(vv)
        cx = c0 - vv
        last = (iv != iv_next) | lane15
        restart = (iv != iv_prev) | lane0
        ivl = iv - win_base
        plsc.addupdate_scatter(acc_v, [ivl], c0, mask=last)
        plsc.addupdate_scatter(acc_v, [ivl], -cx, mask=restart)

    def compute_piece(idxb, valb):
        iv = idxb[pl.ds(0, 16)]
        vreg_step(iv, valb[pl.ds(0, 16)],
                  plsc.load_gather(idxb, [prev_sel]), idxb[pl.ds(1, 16)])

        def vb(j, _):
            for u in range(4):
                b = 64 * j + 16 * u + 16
                vreg_step(idxb[pl.ds(b, 16)], valb[pl.ds(b, 16)],
                          idxb[pl.ds(b - 1, 16)], idxb[pl.ds(b + 1, 16)])
            return _

        lax.fori_loop(0, (NV - 2) // 4, vb, None)

        b = PIECE - 16
        vreg_step(idxb[pl.ds(b, 16)], valb[pl.ds(b, 16)],
                  idxb[pl.ds(b - 1, 16)],
                  plsc.load_gather(idxb, [next_clamp]))

    def round_step(k, _):
        for t in range(2):
            idxb, valb, sem = tbufs[t]

            @pl.when((k < NTEC) & (k % 2 == t))
            def _():
                wait_load(idxb, valb, sem)

                @pl.when(fits)
                def _():
                    compute_piece(idxb, valb)

                @pl.when(jnp.logical_not(fits))
                def _():
                    pltpu.sync_copy(valb, acc.at[idxb], add=True)

                @pl.when(k + 2 < NTEC)
                def _():
                    start_load(NSTR + k + 2, idxb, valb, sem)

        for t in range(4):
            idxb, valb, lsem, csem = sbufs[t]

            @pl.when((k < NSTR) & (k % 4 == t))
            def _():
                wait_load(idxb, valb, lsem)
                pltpu.async_copy(valb, acc.at[idxb], csem, add=True)

            @pl.when((k + 2 < NSTR) & ((k + 2) % 4 == t))
            def _():
                @pl.when(k >= 2)
                def _():
                    pltpu.make_async_copy(valb, acc.at[idxb], csem).wait()

                start_load(k + 2, idxb, valb, lsem)

        return _

    lax.fori_loop(0, NROUND, round_step, None)

    for j in range(NSTR - 4, NSTR):
        idxb, valb, _, csem = sbufs[j % 4]
        pltpu.make_async_copy(valb, acc.at[idxb], csem).wait()

    plsc.subcore_barrier()

    for r in range(WCH):
        @pl.when(fits & (k_lo + r <= k_hi))
        def _():
            def ib(i, _):
                iotabuf[pl.ds(16 * i, 16)] = (iota16 + win_base
                                              + (r * CCH + 16 * i))
                return _

            lax.fori_loop(0, CCH // 16, ib, None)
            pltpu.sync_copy(acc_v.at[pl.ds(r * CCH, CCH)], acc.at[iotabuf],
                            add=True)

    plsc.subcore_barrier()

    pltpu.sync_copy(acc.at[pl.ds(s * ZCH, ZCH)],
                    part_hbm.at[pl.ds(c * NMP + s * ZCH, ZCH)])


def _combine_kernel(part_hbm, out_hbm, bufa, bufb):
    c = lax.axis_index("c")
    s = lax.axis_index("s")
    w = c * NS + s
    ch = NMP // (NC * NS)
    base = w * ch
    pltpu.sync_copy(part_hbm.at[pl.ds(base, ch)], bufa)
    pltpu.sync_copy(part_hbm.at[pl.ds(NMP + base, ch)], bufb)

    def add_body(j, _):
        sl = pl.ds(16 * j, 16)
        bufa[sl] = bufa[sl] + bufb[sl]
        return _

    lax.fori_loop(0, ch // 16, add_body, None)
    pltpu.sync_copy(bufa, out_hbm.at[pl.ds(base, ch)])


def kernel(indices, per_atom_property):
    mesh = plsc.VectorSubcoreMesh(core_axis_name="c", subcore_axis_name="s")

    partials = pl.kernel(
        _partials_kernel,
        out_type=jax.ShapeDtypeStruct((NC * NMP,), jnp.float32),
        mesh=mesh,
        compiler_params=pltpu.CompilerParams(needs_layout_passes=False),
        scratch_types=(
            [pltpu.VMEM((PIECE,), jnp.int32), pltpu.VMEM((PIECE,), jnp.float32)] * 6
            + [
                pltpu.VMEM((CCH,), jnp.int32),
                pltpu.VMEM((16,), jnp.int32),
                pltpu.VMEM((WCH * CCH,), jnp.float32),
                pltpu.VMEM_SHARED((NMP,), jnp.float32),
            ]
            + [pltpu.SemaphoreType.DMA] * 10
        ),
    )(indices, per_atom_property)

    out = pl.kernel(
        _combine_kernel,
        out_type=jax.ShapeDtypeStruct((NMP,), jnp.float32),
        mesh=mesh,
        scratch_types=[
            pltpu.VMEM((NMP // (NC * NS),), jnp.float32),
            pltpu.VMEM((NMP // (NC * NS),), jnp.float32),
        ],
    )(partials)

    return out[:NM]


# speedup vs baseline: 1.0554x; 1.0057x over previous
import jax
import jax.numpy as jnp
from jax import lax
from jax.experimental import pallas as pl
from jax.experimental.pallas import tpu as pltpu
from jax.experimental.pallas import tpu_sc as plsc

NA = 6_400_000
NM = 100_000
NMP = 100_352
NC = 2
NS = 16
APW = NA // (NC * NS)
PIECE = 4_000
NPIECE = APW // PIECE
NSTR = 24
NTEC = NPIECE - NSTR
NROUND = max(NSTR, NTEC)
ZCH = NMP // NS
CCH = NMP // 32
WCH = 4
NV = PIECE // 16


def _partials_kernel(idx_hbm, val_hbm, part_hbm,
                     tidx0, tval0, tidx1, tval1,
                     sidx0, sval0, sidx1, sval1, sidx2, sval2, sidx3, sval3,
                     iotabuf, t16, acc_v, acc,
                     tl0, tl1, sl0, sl1, sl2, sl3, sc0, sc1, sc2, sc3):
    c = lax.axis_index("c")
    s = lax.axis_index("s")
    wid = c * NS + s
    tbufs = ((tidx0, tval0, tl0), (tidx1, tval1, tl1))
    sbufs = ((sidx0, sval0, sl0, sc0), (sidx1, sval1, sl1, sc1),
             (sidx2, sval2, sl2, sc2), (sidx3, sval3, sl3, sc3))

    def start_load(i, idxb, valb, sem):
        base = wid * APW + i * PIECE
        pltpu.async_copy(idx_hbm.at[pl.ds(base, PIECE)], idxb, sem)
        pltpu.async_copy(val_hbm.at[pl.ds(base, PIECE)], valb, sem)

    def wait_load(idxb, valb, sem):
        pltpu.make_async_copy(idx_hbm.at[pl.ds(0, PIECE)], idxb, sem).wait()
        pltpu.make_async_copy(val_hbm.at[pl.ds(0, PIECE)], valb, sem).wait()

    def zero_body(j, _):
        acc_v[pl.ds(16 * j, 16)] = jnp.zeros((16,), jnp.float32)
        return _

    lax.fori_loop(0, WCH * CCH // 16, zero_body, None)
    pltpu.sync_copy(acc_v.at[pl.ds(0, ZCH)], acc.at[pl.ds(s * ZCH, ZCH)])

    start_load(0, sidx0, sval0, sl0)
    start_load(1, sidx1, sval1, sl1)
    start_load(NSTR + 0, tidx0, tval0, tl0)
    start_load(NSTR + 1, tidx1, tval1, tl1)

    pltpu.sync_copy(idx_hbm.at[pl.ds(wid * APW, 16)], t16)
    m_first = t16[...][0]
    pltpu.sync_copy(idx_hbm.at[pl.ds(wid * APW + APW - 16, 16)], t16)
    m_last = t16[...][15]
    k_lo = m_first // CCH
    k_hi = m_last // CCH
    fits = (k_hi - k_lo) < WCH
    win_base = k_lo * CCH

    iota16 = lax.iota(jnp.int32, 16)
    prev_sel = jnp.maximum(iota16 - 1, 0)
    next_clamp = jnp.minimum(iota16 + (PIECE - 16) + 1, PIECE - 1)
    lane0 = iota16 == 0
    lane15 = iota16 == 15

    def vreg_step(iv, vv, iv_prev, iv_next):
        c0 = plsc.cumsum(vv)
        cx = c0 - vv
        last = (iv != iv_next) | lane15
        restart = (iv != iv_prev) | lane0
        ivl = iv - win_base
        plsc.addupdate_scatter(acc_v, [ivl], c0, mask=last)
        plsc.addupdate_scatter(acc_v, [ivl], -cx, mask=restart)

    def compute_piece(idxb, valb):
        iv = idxb[pl.ds(0, 16)]
        vreg_step(iv, valb[pl.ds(0, 16)],
                  plsc.load_gather(idxb, [prev_sel]), idxb[pl.ds(1, 16)])

        def vb(j, _):
            for u in range(4):
                b = 64 * j + 16 * u + 16
                vreg_step(idxb[pl.ds(b, 16)], valb[pl.ds(b, 16)],
                          idxb[pl.ds(b - 1, 16)], idxb[pl.ds(b + 1, 16)])
            return _

        lax.fori_loop(0, (NV - 2) // 4, vb, None)

        b = PIECE - 16
        vreg_step(idxb[pl.ds(b, 16)], valb[pl.ds(b, 16)],
                  idxb[pl.ds(b - 1, 16)],
                  plsc.load_gather(idxb, [next_clamp]))

    def round_step(k, _):
        for t in range(2):
            idxb, valb, sem = tbufs[t]

            @pl.when((k < NTEC) & (k % 2 == t))
            def _():
                wait_load(idxb, valb, sem)

                @pl.when(fits)
                def _():
                    compute_piece(idxb, valb)

                @pl.when(jnp.logical_not(fits))
                def _():
                    pltpu.sync_copy(valb, acc.at[idxb], add=True)

                @pl.when(k + 2 < NTEC)
                def _():
                    start_load(NSTR + k + 2, idxb, valb, sem)

        for t in range(4):
            idxb, valb, lsem, csem = sbufs[t]

            @pl.when((k < NSTR) & (k % 4 == t))
            def _():
                wait_load(idxb, valb, lsem)
                pltpu.async_copy(valb, acc.at[idxb], csem, add=True)

            @pl.when((k + 2 < NSTR) & ((k + 2) % 4 == t))
            def _():
                @pl.when(k >= 2)
                def _():
                    pltpu.make_async_copy(valb, acc.at[idxb], csem).wait()

                start_load(k + 2, idxb, valb, lsem)

        return _

    lax.fori_loop(0, NROUND, round_step, None)

    for j in range(NSTR - 4, NSTR):
        idxb, valb, _, csem = sbufs[j % 4]
        pltpu.make_async_copy(valb, acc.at[idxb], csem).wait()

    plsc.subcore_barrier()

    for r in range(WCH):
        @pl.when(fits & (k_lo + r <= k_hi))
        def _():
            def ib(i, _):
                iotabuf[pl.ds(16 * i, 16)] = (iota16 + win_base
                                              + (r * CCH + 16 * i))
                return _

            lax.fori_loop(0, CCH // 16, ib, None)
            pltpu.sync_copy(acc_v.at[pl.ds(r * CCH, CCH)], acc.at[iotabuf],
                            add=True)

    plsc.subcore_barrier()

    pltpu.sync_copy(acc.at[pl.ds(s * ZCH, ZCH)],
                    part_hbm.at[pl.ds(c * NMP + s * ZCH, ZCH)])


def _combine_kernel(part_hbm, out_hbm, bufa, bufb):
    c = lax.axis_index("c")
    s = lax.axis_index("s")
    w = c * NS + s
    ch = NMP // (NC * NS)
    base = w * ch
    pltpu.sync_copy(part_hbm.at[pl.ds(base, ch)], bufa)
    pltpu.sync_copy(part_hbm.at[pl.ds(NMP + base, ch)], bufb)

    def add_body(j, _):
        sl = pl.ds(16 * j, 16)
        bufa[sl] = bufa[sl] + bufb[sl]
        return _

    lax.fori_loop(0, ch // 16, add_body, None)
    pltpu.sync_copy(bufa, out_hbm.at[pl.ds(base, ch)])


def kernel(indices, per_atom_property):
    mesh = plsc.VectorSubcoreMesh(core_axis_name="c", subcore_axis_name="s")

    partials = pl.kernel(
        _partials_kernel,
        out_type=jax.ShapeDtypeStruct((NC * NMP,), jnp.float32),
        mesh=mesh,
        compiler_params=pltpu.CompilerParams(needs_layout_passes=False),
        scratch_types=(
            [pltpu.VMEM((PIECE,), jnp.int32), pltpu.VMEM((PIECE,), jnp.float32)] * 6
            + [
                pltpu.VMEM((CCH,), jnp.int32),
                pltpu.VMEM((16,), jnp.int32),
                pltpu.VMEM((WCH * CCH,), jnp.float32),
                pltpu.VMEM_SHARED((NMP,), jnp.float32),
            ]
            + [pltpu.SemaphoreType.DMA] * 10
        ),
    )(indices, per_atom_property)

    out = pl.kernel(
        _combine_kernel,
        out_type=jax.ShapeDtypeStruct((NMP,), jnp.float32),
        mesh=mesh,
        scratch_types=[
            pltpu.VMEM((NMP // (NC * NS),), jnp.float32),
            pltpu.VMEM((NMP // (NC * NS),), jnp.float32),
        ],
    )(partials)

    return out[:NM]
